# baseline (speedup 1.0000x reference)
import functools

import jax
import jax.numpy as jnp
from jax import lax
from jax.experimental import pallas as pl
from jax.experimental.pallas import tpu as pltpu
from jax.experimental.pallas import tpu_sc as plsc

N_ENTITY = 1000000
N_RELATION = 1000
EMBED = 64
BATCH = 16384

_L = 16
_CHUNKS = EMBED // _L


_GATHER_DNUMS = lax.GatherDimensionNumbers(
    offset_dims=(), collapsed_slice_dims=(0,), start_index_map=(0,))


def _permute16(v, idx):
    return lax.gather(v, idx[:, None], _GATHER_DNUMS, slice_sizes=(1,),
                      mode=lax.GatherScatterMode.PROMISE_IN_BOUNDS)


def _hsum16(v):
    iota = lax.iota(jnp.int32, _L)
    for sh in (8, 4, 2, 1):
        v = v + _permute16(v, iota ^ sh)
    return v


def _rsqrt16(s):
    s = jnp.maximum(s, 1e-12)
    i = plsc.bitcast(s, jnp.int32)
    i = jnp.int32(0x5F3759DF) - lax.shift_right_logical(i, 1)
    y = plsc.bitcast(i, jnp.float32)
    for _ in range(3):
        y = y * (1.5 - 0.5 * s * y * y)
    return y


def _make_sc_kernel(n_batch, n_workers):
    b_per_w = n_batch // n_workers
    mesh = plsc.VectorSubcoreMesh(core_axis_name="c", subcore_axis_name="s")
    info = plsc.get_sparse_core_info()
    nc = info.num_cores

    @functools.partial(
        pl.kernel,
        mesh=mesh,
        compiler_params=pltpu.CompilerParams(
            needs_layout_passes=False, use_tc_tiling_on_sc=False),
        out_type=jax.ShapeDtypeStruct((n_batch, EMBED), jnp.float32),
        scratch_types=[
            pltpu.VMEM((b_per_w,), jnp.int32),
            pltpu.VMEM((b_per_w,), jnp.int32),
            pltpu.VMEM((b_per_w, EMBED), jnp.float32),
            pltpu.VMEM((b_per_w, EMBED), jnp.float32),
            pltpu.SemaphoreType.DMA,
            pltpu.SemaphoreType.DMA,
        ],
    )
    def translate(src_hbm, rel_idx_hbm, ent_hbm, rel_hbm, out_hbm,
                  idx_s_v, idx_r_v, ent_v, rel_v, sem_e, sem_r):
        wid = lax.axis_index("s") * nc + lax.axis_index("c")
        base = wid * b_per_w
        pltpu.sync_copy(src_hbm.at[pl.ds(base, b_per_w)], idx_s_v)
        pltpu.sync_copy(rel_idx_hbm.at[pl.ds(base, b_per_w)], idx_r_v)
        cp_e = pltpu.async_copy(ent_hbm.at[idx_s_v], ent_v, sem_e)
        cp_r = pltpu.async_copy(rel_hbm.at[idx_r_v], rel_v, sem_r)
        cp_e.wait()
        cp_r.wait()

        def row_body(i, carry):
            e = [ent_v[i, pl.ds(j * _L, _L)] for j in range(_CHUNKS)]
            rl = [rel_v[i, pl.ds(j * _L, _L)] for j in range(_CHUNKS)]
            sq_e = e[0] * e[0]
            sq_r = rl[0] * rl[0]
            for j in range(1, _CHUNKS):
                sq_e = sq_e + e[j] * e[j]
                sq_r = sq_r + rl[j] * rl[j]
            inv_e = _rsqrt16(_hsum16(sq_e))
            inv_r = _rsqrt16(_hsum16(sq_r))
            for j in range(_CHUNKS):
                ent_v[i, pl.ds(j * _L, _L)] = e[j] * inv_e + rl[j] * inv_r
            return carry

        lax.fori_loop(0, b_per_w, row_body, 0)
        pltpu.sync_copy(ent_v, out_hbm.at[pl.ds(base, b_per_w)])

    return translate


def kernel(source, r, entity_embeddings, relation_embeddings):
    info = plsc.get_sparse_core_info()
    n_workers = info.num_cores * info.num_subcores
    fn = _make_sc_kernel(BATCH, n_workers)
    return fn(source.astype(jnp.int32), r.astype(jnp.int32),
              entity_embeddings, relation_embeddings)
